# baseline (speedup 1.0000x reference)
import functools

import jax
import jax.numpy as jnp
from jax import lax
from jax.experimental import pallas as pl
from jax.experimental.pallas import tpu as pltpu
from jax.experimental.pallas import tpu_sc as plsc

N = 100000
E = 1600000
NP = 100352
RPT = NP // 16
HID = 64
MLP_HID = 128
K = 15

EP = 1612800
EB = 800
BN3 = 2000
BN1 = 6272

_MESH = dict(core_axis_name="c", subcore_axis_name="s")
_SC_PARAMS = dict(
    compiler_params=pltpu.CompilerParams(use_tc_tiling_on_sc=False,
                                         skip_device_barrier=True))


def _deg_body(dst_hbm, zeros_hbm, out_hbm, dst0, dst1, ones_v, acc_sh,
              s0, s1):
    c = lax.axis_index("c")
    s = lax.axis_index("s")

    def fill(j, carry):
        ones_v[pl.ds(j * 16, 16)] = jnp.full((16,), 1.0, jnp.float32)
        return carry

    lax.fori_loop(0, EB // 16, fill, 0)
    pltpu.sync_copy(zeros_hbm.at[pl.ds(s * RPT, RPT)],
                    acc_sh.at[pl.ds(s * RPT, RPT)])
    plsc.subcore_barrier()
    wid = c * 16 + s
    ept = EP // 32
    nblk = ept // EB
    base0 = wid * ept
    bufs = (dst0, dst1)
    sems = (s0, s1)

    pltpu.async_copy(dst_hbm.at[pl.ds(base0, EB)], dst0, s0)

    def blk(i, carry):
        def one(b):
            @pl.when(i + 1 < nblk)
            def _():
                pltpu.async_copy(
                    dst_hbm.at[pl.ds(base0 + (i + 1) * EB, EB)],
                    bufs[1 - b], sems[1 - b])

            pltpu.make_async_copy(
                dst_hbm.at[pl.ds(base0, EB)], bufs[b], sems[b]).wait()
            pltpu.sync_copy(ones_v, acc_sh.at[bufs[b]], add=True)

        lax.cond(i % 2 == 0, lambda: one(0), lambda: one(1))
        return carry

    lax.fori_loop(0, nblk, blk, 0)
    plsc.subcore_barrier()
    pltpu.sync_copy(acc_sh.at[pl.ds(s * RPT, RPT)],
                    out_hbm.at[c, pl.ds(s * RPT, RPT)])


def _sc_deg(dst, zeros1):
    fn = pl.kernel(
        _deg_body,
        out_type=jax.ShapeDtypeStruct((2, NP), jnp.float32),
        mesh=plsc.VectorSubcoreMesh(**_MESH),
        scratch_types=[
            pltpu.VMEM((EB,), jnp.int32),
            pltpu.VMEM((EB,), jnp.int32),
            pltpu.VMEM((EB,), jnp.float32),
            pltpu.VMEM_SHARED((NP,), jnp.float32),
            pltpu.SemaphoreType.DMA,
            pltpu.SemaphoreType.DMA,
        ],
        **_SC_PARAMS,
    )
    return fn(dst, zeros1)


def _edge_loop(EB, nblk, base0, idx_mul, idx_off, src_hbm, dst_hbm, tab_hbm,
               srcb, dstb, rowsb, isems, gsems, acc_sh, db=True):
    def issue_idx(i, b):
        pltpu.async_copy(src_hbm.at[pl.ds(base0 + i * EB, EB)], srcb[b],
                         isems[b])
        pltpu.async_copy(dst_hbm.at[pl.ds(base0 + i * EB, EB)], dstb[b],
                         isems[b])

    def wait_idx(b):
        pltpu.make_async_copy(src_hbm.at[pl.ds(base0, EB)], srcb[b],
                              isems[b]).wait()
        pltpu.make_async_copy(dst_hbm.at[pl.ds(base0, EB)], dstb[b],
                              isems[b]).wait()

    def adjust(b):
        if idx_mul == 1 and idx_off is None:
            return

        def adj(j, carry):
            v = srcb[b][pl.ds(j * 16, 16)]
            if idx_mul != 1:
                v = v * idx_mul
            if idx_off is not None:
                v = v + jnp.full((16,), 1, jnp.int32) * idx_off
            srcb[b][pl.ds(j * 16, 16)] = v
            return carry

        lax.fori_loop(0, EB // 16, adj, 0)

    def issue_gather(b):
        pltpu.async_copy(tab_hbm.at[srcb[b]], rowsb[b], gsems[b])

    def wait_gather(b):
        pltpu.make_async_copy(tab_hbm.at[srcb[b]], rowsb[b], gsems[b]).wait()

    def issue_idx_dyn(i2, b):
        pltpu.async_copy(src_hbm.at[pl.ds(base0 + i2 * EB, EB)], srcb[b],
                         isems[b])
        pltpu.async_copy(dst_hbm.at[pl.ds(base0 + i2 * EB, EB)], dstb[b],
                         isems[b])

    if not db:
        def sblk(i, carry):
            issue_idx_dyn(i, 0)
            wait_idx(0)
            adjust(0)
            issue_gather(0)
            wait_gather(0)
            pltpu.sync_copy(rowsb[0], acc_sh.at[dstb[0]], add=True)
            return carry

        lax.fori_loop(0, nblk, sblk, 0)
        return

    issue_idx(0, 0)
    wait_idx(0)
    adjust(0)
    issue_gather(0)
    if nblk > 1:
        issue_idx(1, 1)

    def blk(i, carry):
        def one(b):
            nb = 1 - b

            @pl.when(i + 1 < nblk)
            def _():
                wait_idx(nb)
                adjust(nb)

            wait_gather(b)

            @pl.when(i + 1 < nblk)
            def _():
                issue_gather(nb)

            pltpu.sync_copy(rowsb[b], acc_sh.at[dstb[b]], add=True)

            @pl.when(i + 2 < nblk)
            def _():
                issue_idx_dyn(i + 2, b)

        lax.cond(i % 2 == 0, lambda: one(0), lambda: one(1))
        return carry

    lax.fori_loop(0, nblk, blk, 0)


def _agg1_body(src_hbm, dst_hbm, xd_hbm, zeros_hbm, out_hbm,
               src0, src1, dst0, dst1, rows0, rows1, acc_sh,
               i0, i1, g0, g1):
    c = lax.axis_index("c")
    s = lax.axis_index("s")
    pltpu.sync_copy(zeros_hbm.at[pl.ds(s * RPT, RPT)],
                    acc_sh.at[pl.ds(s * RPT, RPT)])
    plsc.subcore_barrier()
    wid = c * 16 + s
    ept = EP // 32
    _edge_loop(EB, ept // EB, wid * ept, 1, None, src_hbm, dst_hbm, xd_hbm,
               (src0, src1), (dst0, dst1), (rows0, rows1),
               (i0, i1), (g0, g1), acc_sh, db=True)
    plsc.subcore_barrier()
    pltpu.sync_copy(acc_sh.at[pl.ds(s * RPT, RPT)],
                    out_hbm.at[c, pl.ds(s * RPT, RPT)])


def _sc_agg1(src, dst, xd, zeros16):
    fn = pl.kernel(
        _agg1_body,
        out_type=jax.ShapeDtypeStruct((2, NP, 16), jnp.float32),
        mesh=plsc.VectorSubcoreMesh(**_MESH),
        scratch_types=[
            pltpu.VMEM((EB,), jnp.int32),
            pltpu.VMEM((EB,), jnp.int32),
            pltpu.VMEM((EB,), jnp.int32),
            pltpu.VMEM((EB,), jnp.int32),
            pltpu.VMEM((EB, 16), jnp.float32),
            pltpu.VMEM((EB, 16), jnp.float32),
            pltpu.VMEM_SHARED((NP, 16), jnp.float32),
            pltpu.SemaphoreType.DMA,
            pltpu.SemaphoreType.DMA,
            pltpu.SemaphoreType.DMA,
            pltpu.SemaphoreType.DMA,
        ],
        **_SC_PARAMS,
    )
    return fn(src, dst, xd, zeros16)


def _agg2_body(src_hbm, dst_hbm, u_hbm, zeros_hbm, out_hbm,
               src0, src1, dst0, dst1, rows0, rows1, acc_sh,
               i0, i1, g0, g1):
    c = lax.axis_index("c")
    s = lax.axis_index("s")
    ept = EP // 16

    for p in range(2):
        chunk = c * 2 + p
        pltpu.sync_copy(zeros_hbm.at[pl.ds(s * RPT, RPT)],
                        acc_sh.at[pl.ds(s * RPT, RPT)])
        plsc.subcore_barrier()
        _edge_loop(EB, ept // EB, s * ept, 4, chunk, src_hbm, dst_hbm,
                   u_hbm, (src0, src1), (dst0, dst1), (rows0, rows1),
                   (i0, i1), (g0, g1), acc_sh)
        plsc.subcore_barrier()
        pltpu.sync_copy(acc_sh.at[pl.ds(s * RPT, RPT)],
                        out_hbm.at[chunk, pl.ds(s * RPT, RPT)])
        plsc.subcore_barrier()


def _sc_agg2(src, dst, u_flat, zeros16):
    fn = pl.kernel(
        _agg2_body,
        out_type=jax.ShapeDtypeStruct((4, NP, 16), jnp.float32),
        mesh=plsc.VectorSubcoreMesh(**_MESH),
        scratch_types=[
            pltpu.VMEM((EB,), jnp.int32),
            pltpu.VMEM((EB,), jnp.int32),
            pltpu.VMEM((EB,), jnp.int32),
            pltpu.VMEM((EB,), jnp.int32),
            pltpu.VMEM((EB, 16), jnp.float32),
            pltpu.VMEM((EB, 16), jnp.float32),
            pltpu.VMEM_SHARED((NP, 16), jnp.float32),
            pltpu.SemaphoreType.DMA,
            pltpu.SemaphoreType.DMA,
            pltpu.SemaphoreType.DMA,
            pltpu.SemaphoreType.DMA,
        ],
        **_SC_PARAMS,
    )
    return fn(src, dst, u_flat, zeros16)


def _tc1_body(degp_ref, x_ref, out_ref):
    d = degp_ref[0] + degp_ref[1] + 1.0
    dinv = lax.rsqrt(d)
    xb = x_ref[...]
    out_ref[...] = jnp.concatenate(
        [xb * dinv, dinv, jnp.zeros((BN1, 12), jnp.float32)], axis=1)


def _tc1(degp, x_pad):
    return pl.pallas_call(
        _tc1_body,
        grid=(NP // BN1,),
        in_specs=[
            pl.BlockSpec((2, BN1, 1), lambda i: (0, i, 0)),
            pl.BlockSpec((BN1, 3), lambda i: (i, 0)),
        ],
        out_specs=pl.BlockSpec((BN1, 16), lambda i: (i, 0)),
        out_shape=jax.ShapeDtypeStruct((NP, 16), jnp.float32),
    )(degp.reshape(2, NP, 1), x_pad)


def _tc2_body(xd_ref, aggp_ref, w1_ref, b1_ref, out_ref):
    xb = xd_ref[...]
    ag = aggp_ref[0] + aggp_ref[1] + xb
    dinv = xb[:, 3:4]
    z = jnp.dot(ag * dinv, w1_ref[...], preferred_element_type=jnp.float32)
    z = z + b1_ref[...]
    h1 = jnp.where(z > 0, z, jnp.exp(z) - 1.0)
    out_ref[...] = h1 * dinv


def _tc2(xd, agg1p, W1p, b1):
    return pl.pallas_call(
        _tc2_body,
        grid=(NP // BN1,),
        in_specs=[
            pl.BlockSpec((BN1, 16), lambda i: (i, 0)),
            pl.BlockSpec((2, BN1, 16), lambda i: (0, i, 0)),
            pl.BlockSpec((16, HID), lambda i: (0, 0)),
            pl.BlockSpec((1, HID), lambda i: (0, 0)),
        ],
        out_specs=pl.BlockSpec((BN1, HID), lambda i: (i, 0)),
        out_shape=jax.ShapeDtypeStruct((NP, HID), jnp.float32),
    )(xd, agg1p, W1p, b1.reshape(1, HID))


def _tc3_body(agg2_ref, u_ref, xd_ref, w2_ref, b2_ref, wm1_ref, bm1_ref,
              wm2_ref, bm2_ref, out_ref):
    u = u_ref[...]
    dinv = xd_ref[:, 3:4]
    z2 = jnp.dot(u * dinv, w2_ref[...], preferred_element_type=jnp.float32)
    for c0 in range(4):
        z2 = z2 + jnp.dot(agg2_ref[c0] * dinv,
                          w2_ref[16 * c0:16 * (c0 + 1), :],
                          preferred_element_type=jnp.float32)
    z2 = z2 + b2_ref[...]
    h2 = jnp.where(z2 > 0, z2, jnp.exp(z2) - 1.0)
    zm = jnp.dot(h2, wm1_ref[...], preferred_element_type=jnp.float32)
    zm = zm + bm1_ref[...]
    hm = jnp.where(zm > 0, zm, jnp.exp(zm) - 1.0)
    H = jnp.dot(hm, wm2_ref[...], preferred_element_type=jnp.float32)
    H = H + bm2_ref[...]
    m = jnp.max(H, axis=1, keepdims=True)
    e = jnp.exp(H - m)
    S = e / jnp.sum(e, axis=1, keepdims=True)
    out_ref[...] = S[:, :K]


def _tc3(agg2, u, xd, W2, b2, Wm1, bm1, Wm2p, bm2p):
    return pl.pallas_call(
        _tc3_body,
        grid=(N // BN3,),
        in_specs=[
            pl.BlockSpec((4, BN3, 16), lambda i: (0, i, 0)),
            pl.BlockSpec((BN3, HID), lambda i: (i, 0)),
            pl.BlockSpec((BN3, 16), lambda i: (i, 0)),
            pl.BlockSpec((HID, HID), lambda i: (0, 0)),
            pl.BlockSpec((1, HID), lambda i: (0, 0)),
            pl.BlockSpec((HID, MLP_HID), lambda i: (0, 0)),
            pl.BlockSpec((1, MLP_HID), lambda i: (0, 0)),
            pl.BlockSpec((MLP_HID, 16), lambda i: (0, 0)),
            pl.BlockSpec((1, 16), lambda i: (0, 0)),
        ],
        out_specs=pl.BlockSpec((BN3, K), lambda i: (i, 0)),
        out_shape=jax.ShapeDtypeStruct((N, K), jnp.float32),
    )(agg2, u, xd, W2, b2.reshape(1, HID), Wm1, bm1.reshape(1, MLP_HID),
      Wm2p, bm2p.reshape(1, 16))


def kernel(x, edge_index, W1, b1, W2, b2, Wm1, bm1, Wm2, bm2):
    x_pad = jnp.zeros((NP, 3), jnp.float32).at[:N].set(x)
    W1p = jnp.zeros((16, HID), jnp.float32).at[:3].set(W1)
    Wm2p = jnp.zeros((MLP_HID, 16), jnp.float32).at[:, :K].set(Wm2)
    bm2p = jnp.concatenate([bm2, jnp.full((1,), -1e30, jnp.float32)])
    zeros1 = jnp.zeros((NP,), jnp.float32)
    zeros16 = jnp.zeros((NP, 16), jnp.float32)
    spread = jnp.arange(EP, dtype=jnp.int32) & 255
    src = spread.at[:E].set(edge_index[0])
    dst = (N + spread).at[:E].set(edge_index[1])
    degp = _sc_deg(dst, zeros1)
    xd = _tc1(degp, x_pad)
    agg1p = _sc_agg1(src, dst, xd, zeros16)
    u = _tc2(xd, agg1p, W1p, b1)
    agg2 = _sc_agg2(src, dst, u.reshape(4 * NP, 16), zeros16)
    return _tc3(agg2, u, xd, W2, b2, Wm1, bm1, Wm2p, bm2p)


# speedup vs baseline: 1.0617x; 1.0617x over previous
import functools

import jax
import jax.numpy as jnp
from jax import lax
from jax.experimental import pallas as pl
from jax.experimental.pallas import tpu as pltpu
from jax.experimental.pallas import tpu_sc as plsc

N = 100000
E = 1600000
NP = 100352
RPT = NP // 16
HID = 64
MLP_HID = 128
K = 15

EBA = 1000
EBB = 400
EBC = 800
BN3 = 2000
BN1 = 6272

_MESH = dict(core_axis_name="c", subcore_axis_name="s")
_SC_PARAMS = dict(
    compiler_params=pltpu.CompilerParams(use_tc_tiling_on_sc=False,
                                         skip_device_barrier=True))


def _deg_body(dst_hbm, zeros_hbm, out_hbm, dst0, dst1, ones_v, acc_sh,
              s0, s1):
    c = lax.axis_index("c")
    s = lax.axis_index("s")

    def fill(j, carry):
        ones_v[pl.ds(j * 16, 16)] = jnp.full((16,), 1.0, jnp.float32)
        return carry

    lax.fori_loop(0, EBA // 16, fill, 0)
    pltpu.sync_copy(zeros_hbm.at[pl.ds(s * RPT, RPT)],
                    acc_sh.at[pl.ds(s * RPT, RPT)])
    plsc.subcore_barrier()
    wid = c * 16 + s
    ept = E // 32
    nblk = ept // EBA
    base0 = wid * ept
    bufs = (dst0, dst1)
    sems = (s0, s1)

    pltpu.async_copy(dst_hbm.at[pl.ds(base0, EBA)], dst0, s0)

    def blk(i, carry):
        def one(b):
            @pl.when(i + 1 < nblk)
            def _():
                pltpu.async_copy(
                    dst_hbm.at[pl.ds(base0 + (i + 1) * EBA, EBA)],
                    bufs[1 - b], sems[1 - b])

            pltpu.make_async_copy(
                dst_hbm.at[pl.ds(base0, EBA)], bufs[b], sems[b]).wait()
            pltpu.sync_copy(ones_v, acc_sh.at[bufs[b]], add=True)

        lax.cond(i % 2 == 0, lambda: one(0), lambda: one(1))
        return carry

    lax.fori_loop(0, nblk, blk, 0)
    plsc.subcore_barrier()
    pltpu.sync_copy(acc_sh.at[pl.ds(s * RPT, RPT)],
                    out_hbm.at[c, pl.ds(s * RPT, RPT)])


def _sc_deg(dst, zeros1):
    fn = pl.kernel(
        _deg_body,
        out_type=jax.ShapeDtypeStruct((2, NP), jnp.float32),
        mesh=plsc.VectorSubcoreMesh(**_MESH),
        scratch_types=[
            pltpu.VMEM((EBA,), jnp.int32),
            pltpu.VMEM((EBA,), jnp.int32),
            pltpu.VMEM((EBA,), jnp.float32),
            pltpu.VMEM_SHARED((NP,), jnp.float32),
            pltpu.SemaphoreType.DMA,
            pltpu.SemaphoreType.DMA,
        ],
        **_SC_PARAMS,
    )
    return fn(dst, zeros1)


def _edge_loop(EB, nblk, base0, idx_mul, idx_off, src_hbm, dst_hbm, tab_hbm,
               srcb, dstb, rowsb, isems, gsems, acc_sh, db=True):
    def issue_idx(i, b):
        pltpu.async_copy(src_hbm.at[pl.ds(base0 + i * EB, EB)], srcb[b],
                         isems[b])
        pltpu.async_copy(dst_hbm.at[pl.ds(base0 + i * EB, EB)], dstb[b],
                         isems[b])

    def wait_idx(b):
        pltpu.make_async_copy(src_hbm.at[pl.ds(base0, EB)], srcb[b],
                              isems[b]).wait()
        pltpu.make_async_copy(dst_hbm.at[pl.ds(base0, EB)], dstb[b],
                              isems[b]).wait()

    def adjust(b):
        if idx_mul == 1 and idx_off is None:
            return

        def adj(j, carry):
            v = srcb[b][pl.ds(j * 16, 16)]
            if idx_mul != 1:
                v = v * idx_mul
            if idx_off is not None:
                v = v + jnp.full((16,), 1, jnp.int32) * idx_off
            srcb[b][pl.ds(j * 16, 16)] = v
            return carry

        lax.fori_loop(0, EB // 16, adj, 0)

    def issue_gather(b):
        pltpu.async_copy(tab_hbm.at[srcb[b]], rowsb[b], gsems[b])

    def wait_gather(b):
        pltpu.make_async_copy(tab_hbm.at[srcb[b]], rowsb[b], gsems[b]).wait()

    def issue_idx_dyn(i2, b):
        pltpu.async_copy(src_hbm.at[pl.ds(base0 + i2 * EB, EB)], srcb[b],
                         isems[b])
        pltpu.async_copy(dst_hbm.at[pl.ds(base0 + i2 * EB, EB)], dstb[b],
                         isems[b])

    if not db:
        def sblk(i, carry):
            issue_idx_dyn(i, 0)
            wait_idx(0)
            adjust(0)
            issue_gather(0)
            wait_gather(0)
            pltpu.sync_copy(rowsb[0], acc_sh.at[dstb[0]], add=True)
            return carry

        lax.fori_loop(0, nblk, sblk, 0)
        return

    issue_idx(0, 0)
    wait_idx(0)
    adjust(0)
    issue_gather(0)
    if nblk > 1:
        issue_idx(1, 1)

    def blk(i, carry):
        def one(b):
            nb = 1 - b

            @pl.when(i + 1 < nblk)
            def _():
                wait_idx(nb)
                adjust(nb)

            wait_gather(b)

            @pl.when(i + 1 < nblk)
            def _():
                issue_gather(nb)

            pltpu.sync_copy(rowsb[b], acc_sh.at[dstb[b]], add=True)

            @pl.when(i + 2 < nblk)
            def _():
                issue_idx_dyn(i + 2, b)

        lax.cond(i % 2 == 0, lambda: one(0), lambda: one(1))
        return carry

    lax.fori_loop(0, nblk, blk, 0)


def _agg1_body(src_hbm, dst_hbm, xd_hbm, zeros_hbm, out_hbm,
               src0, src1, dst0, dst1, rows0, rows1, acc_sh,
               i0, i1, g0, g1):
    c = lax.axis_index("c")
    s = lax.axis_index("s")
    pltpu.sync_copy(zeros_hbm.at[pl.ds(s * RPT, RPT)],
                    acc_sh.at[pl.ds(s * RPT, RPT)])
    plsc.subcore_barrier()
    wid = c * 16 + s
    ept = E // 32
    _edge_loop(EBB, ept // EBB, wid * ept, 1, None, src_hbm, dst_hbm, xd_hbm,
               (src0, src1), (dst0, dst1), (rows0, rows1),
               (i0, i1), (g0, g1), acc_sh, db=True)
    plsc.subcore_barrier()
    pltpu.sync_copy(acc_sh.at[pl.ds(s * RPT, RPT)],
                    out_hbm.at[c, pl.ds(s * RPT, RPT)])


def _sc_agg1(src, dst, xd, zeros16):
    fn = pl.kernel(
        _agg1_body,
        out_type=jax.ShapeDtypeStruct((2, NP, 16), jnp.float32),
        mesh=plsc.VectorSubcoreMesh(**_MESH),
        scratch_types=[
            pltpu.VMEM((EBB,), jnp.int32),
            pltpu.VMEM((EBB,), jnp.int32),
            pltpu.VMEM((EBB,), jnp.int32),
            pltpu.VMEM((EBB,), jnp.int32),
            pltpu.VMEM((EBB, 16), jnp.float32),
            pltpu.VMEM((EBB, 16), jnp.float32),
            pltpu.VMEM_SHARED((NP, 16), jnp.float32),
            pltpu.SemaphoreType.DMA,
            pltpu.SemaphoreType.DMA,
            pltpu.SemaphoreType.DMA,
            pltpu.SemaphoreType.DMA,
        ],
        **_SC_PARAMS,
    )
    return fn(src, dst, xd, zeros16)


def _agg2_body(src_hbm, dst_hbm, u_hbm, zeros_hbm, out_hbm,
               src0, src1, dst0, dst1, rows0, rows1, acc_sh,
               i0, i1, g0, g1):
    c = lax.axis_index("c")
    s = lax.axis_index("s")
    ept = E // 16

    for p in range(2):
        chunk = c * 2 + p
        pltpu.sync_copy(zeros_hbm.at[pl.ds(s * RPT, RPT)],
                        acc_sh.at[pl.ds(s * RPT, RPT)])
        plsc.subcore_barrier()
        _edge_loop(EBC, ept // EBC, s * ept, 4, chunk, src_hbm, dst_hbm,
                   u_hbm, (src0, src1), (dst0, dst1), (rows0, rows1),
                   (i0, i1), (g0, g1), acc_sh)
        plsc.subcore_barrier()
        pltpu.sync_copy(acc_sh.at[pl.ds(s * RPT, RPT)],
                        out_hbm.at[chunk, pl.ds(s * RPT, RPT)])
        plsc.subcore_barrier()


def _sc_agg2(src, dst, u_flat, zeros16):
    fn = pl.kernel(
        _agg2_body,
        out_type=jax.ShapeDtypeStruct((4, NP, 16), jnp.float32),
        mesh=plsc.VectorSubcoreMesh(**_MESH),
        scratch_types=[
            pltpu.VMEM((EBC,), jnp.int32),
            pltpu.VMEM((EBC,), jnp.int32),
            pltpu.VMEM((EBC,), jnp.int32),
            pltpu.VMEM((EBC,), jnp.int32),
            pltpu.VMEM((EBC, 16), jnp.float32),
            pltpu.VMEM((EBC, 16), jnp.float32),
            pltpu.VMEM_SHARED((NP, 16), jnp.float32),
            pltpu.SemaphoreType.DMA,
            pltpu.SemaphoreType.DMA,
            pltpu.SemaphoreType.DMA,
            pltpu.SemaphoreType.DMA,
        ],
        **_SC_PARAMS,
    )
    return fn(src, dst, u_flat, zeros16)


def _tc1_body(degp_ref, x_ref, out_ref):
    d = degp_ref[0] + degp_ref[1] + 1.0
    dinv = lax.rsqrt(d)
    xb = x_ref[...]
    out_ref[...] = jnp.concatenate(
        [xb * dinv, dinv, jnp.zeros((BN1, 12), jnp.float32)], axis=1)


def _tc1(degp, x_pad):
    return pl.pallas_call(
        _tc1_body,
        grid=(NP // BN1,),
        in_specs=[
            pl.BlockSpec((2, BN1, 1), lambda i: (0, i, 0)),
            pl.BlockSpec((BN1, 3), lambda i: (i, 0)),
        ],
        out_specs=pl.BlockSpec((BN1, 16), lambda i: (i, 0)),
        out_shape=jax.ShapeDtypeStruct((NP, 16), jnp.float32),
    )(degp.reshape(2, NP, 1), x_pad)


def _tc2_body(xd_ref, aggp_ref, w1_ref, b1_ref, out_ref):
    xb = xd_ref[...]
    ag = aggp_ref[0] + aggp_ref[1] + xb
    dinv = xb[:, 3:4]
    z = jnp.dot(ag * dinv, w1_ref[...], preferred_element_type=jnp.float32)
    z = z + b1_ref[...]
    h1 = jnp.where(z > 0, z, jnp.exp(z) - 1.0)
    out_ref[...] = h1 * dinv


def _tc2(xd, agg1p, W1p, b1):
    return pl.pallas_call(
        _tc2_body,
        grid=(NP // BN1,),
        in_specs=[
            pl.BlockSpec((BN1, 16), lambda i: (i, 0)),
            pl.BlockSpec((2, BN1, 16), lambda i: (0, i, 0)),
            pl.BlockSpec((16, HID), lambda i: (0, 0)),
            pl.BlockSpec((1, HID), lambda i: (0, 0)),
        ],
        out_specs=pl.BlockSpec((BN1, HID), lambda i: (i, 0)),
        out_shape=jax.ShapeDtypeStruct((NP, HID), jnp.float32),
    )(xd, agg1p, W1p, b1.reshape(1, HID))


def _tc3_body(agg2_ref, u_ref, xd_ref, w2_ref, b2_ref, wm1_ref, bm1_ref,
              wm2_ref, bm2_ref, out_ref):
    u = u_ref[...]
    dinv = xd_ref[:, 3:4]
    z2 = jnp.dot(u * dinv, w2_ref[...], preferred_element_type=jnp.float32)
    for c0 in range(4):
        z2 = z2 + jnp.dot(agg2_ref[c0] * dinv,
                          w2_ref[16 * c0:16 * (c0 + 1), :],
                          preferred_element_type=jnp.float32)
    z2 = z2 + b2_ref[...]
    h2 = jnp.where(z2 > 0, z2, jnp.exp(z2) - 1.0)
    zm = jnp.dot(h2, wm1_ref[...], preferred_element_type=jnp.float32)
    zm = zm + bm1_ref[...]
    hm = jnp.where(zm > 0, zm, jnp.exp(zm) - 1.0)
    H = jnp.dot(hm, wm2_ref[...], preferred_element_type=jnp.float32)
    H = H + bm2_ref[...]
    m = jnp.max(H, axis=1, keepdims=True)
    e = jnp.exp(H - m)
    S = e / jnp.sum(e, axis=1, keepdims=True)
    out_ref[...] = S[:, :K]


def _tc3(agg2, u, xd, W2, b2, Wm1, bm1, Wm2p, bm2p):
    return pl.pallas_call(
        _tc3_body,
        grid=(N // BN3,),
        in_specs=[
            pl.BlockSpec((4, BN3, 16), lambda i: (0, i, 0)),
            pl.BlockSpec((BN3, HID), lambda i: (i, 0)),
            pl.BlockSpec((BN3, 16), lambda i: (i, 0)),
            pl.BlockSpec((HID, HID), lambda i: (0, 0)),
            pl.BlockSpec((1, HID), lambda i: (0, 0)),
            pl.BlockSpec((HID, MLP_HID), lambda i: (0, 0)),
            pl.BlockSpec((1, MLP_HID), lambda i: (0, 0)),
            pl.BlockSpec((MLP_HID, 16), lambda i: (0, 0)),
            pl.BlockSpec((1, 16), lambda i: (0, 0)),
        ],
        out_specs=pl.BlockSpec((BN3, K), lambda i: (i, 0)),
        out_shape=jax.ShapeDtypeStruct((N, K), jnp.float32),
    )(agg2, u, xd, W2, b2.reshape(1, HID), Wm1, bm1.reshape(1, MLP_HID),
      Wm2p, bm2p.reshape(1, 16))


def kernel(x, edge_index, W1, b1, W2, b2, Wm1, bm1, Wm2, bm2):
    x_pad = jnp.zeros((NP, 3), jnp.float32).at[:N].set(x)
    W1p = jnp.zeros((16, HID), jnp.float32).at[:3].set(W1)
    Wm2p = jnp.zeros((MLP_HID, 16), jnp.float32).at[:, :K].set(Wm2)
    bm2p = jnp.concatenate([bm2, jnp.full((1,), -1e30, jnp.float32)])
    zeros1 = jnp.zeros((NP,), jnp.float32)
    zeros16 = jnp.zeros((NP, 16), jnp.float32)
    src = edge_index[0]
    dst = edge_index[1]
    degp = _sc_deg(dst, zeros1)
    xd = _tc1(degp, x_pad)
    agg1p = _sc_agg1(src, dst, xd, zeros16)
    u = _tc2(xd, agg1p, W1p, b1)
    agg2 = _sc_agg2(src, dst, u.reshape(4 * NP, 16), zeros16)
    return _tc3(agg2, u, xd, W2, b2, Wm1, bm1, Wm2p, bm2p)


# speedup vs baseline: 1.0886x; 1.0254x over previous
import functools

import jax
import jax.numpy as jnp
from jax import lax
from jax.experimental import pallas as pl
from jax.experimental.pallas import tpu as pltpu
from jax.experimental.pallas import tpu_sc as plsc

N = 100000
E = 1600000
NP = 100352
RPT = NP // 16
HID = 64
MLP_HID = 128
K = 15

EBA = 1000
EBB = 400
EBC = 800
BN3 = 2000
BN1 = 6272

_MESH = dict(core_axis_name="c", subcore_axis_name="s")
_SC_PARAMS = dict(
    compiler_params=pltpu.CompilerParams(use_tc_tiling_on_sc=False,
                                         skip_device_barrier=True))


def _deg_body(edge_hbm, zeros_hbm, out_hbm, dst0, dst1, ones_v, acc_sh,
              s0, s1):
    c = lax.axis_index("c")
    s = lax.axis_index("s")

    def fill(j, carry):
        ones_v[pl.ds(j * 16, 16)] = jnp.full((16,), 1.0, jnp.float32)
        return carry

    lax.fori_loop(0, EBA // 16, fill, 0)
    pltpu.sync_copy(zeros_hbm.at[pl.ds(s * RPT, RPT)],
                    acc_sh.at[pl.ds(s * RPT, RPT)])
    plsc.subcore_barrier()
    wid = c * 16 + s
    ept = E // 32
    nblk = ept // EBA
    base0 = wid * ept
    bufs = (dst0, dst1)
    sems = (s0, s1)

    pltpu.async_copy(edge_hbm.at[1, pl.ds(base0, EBA)], dst0, s0)

    def blk(i, carry):
        def one(b):
            @pl.when(i + 1 < nblk)
            def _():
                pltpu.async_copy(
                    edge_hbm.at[1, pl.ds(base0 + (i + 1) * EBA, EBA)],
                    bufs[1 - b], sems[1 - b])

            pltpu.make_async_copy(
                edge_hbm.at[1, pl.ds(base0, EBA)], bufs[b], sems[b]).wait()
            pltpu.sync_copy(ones_v, acc_sh.at[bufs[b]], add=True)

        lax.cond(i % 2 == 0, lambda: one(0), lambda: one(1))
        return carry

    lax.fori_loop(0, nblk, blk, 0)
    plsc.subcore_barrier()
    pltpu.sync_copy(acc_sh.at[pl.ds(s * RPT, RPT)],
                    out_hbm.at[c, pl.ds(s * RPT, RPT)])


def _sc_deg(edge_index, zeros1):
    fn = pl.kernel(
        _deg_body,
        out_type=jax.ShapeDtypeStruct((2, NP), jnp.float32),
        mesh=plsc.VectorSubcoreMesh(**_MESH),
        scratch_types=[
            pltpu.VMEM((EBA,), jnp.int32),
            pltpu.VMEM((EBA,), jnp.int32),
            pltpu.VMEM((EBA,), jnp.float32),
            pltpu.VMEM_SHARED((NP,), jnp.float32),
            pltpu.SemaphoreType.DMA,
            pltpu.SemaphoreType.DMA,
        ],
        **_SC_PARAMS,
    )
    return fn(edge_index, zeros1)


def _edge_loop(EB, nblk, base0, idx_mul, idx_off, edge_hbm, tab_hbm,
               srcb, dstb, rowsb, isems, gsems, acc_sh, db=True):
    def issue_idx(i, b):
        pltpu.async_copy(edge_hbm.at[0, pl.ds(base0 + i * EB, EB)], srcb[b],
                         isems[b])
        pltpu.async_copy(edge_hbm.at[1, pl.ds(base0 + i * EB, EB)], dstb[b],
                         isems[b])

    def wait_idx(b):
        pltpu.make_async_copy(edge_hbm.at[0, pl.ds(base0, EB)], srcb[b],
                              isems[b]).wait()
        pltpu.make_async_copy(edge_hbm.at[1, pl.ds(base0, EB)], dstb[b],
                              isems[b]).wait()

    def adjust(b):
        if idx_mul == 1 and idx_off is None:
            return

        def adj(j, carry):
            v = srcb[b][pl.ds(j * 16, 16)]
            if idx_mul != 1:
                v = v * idx_mul
            if idx_off is not None:
                v = v + jnp.full((16,), 1, jnp.int32) * idx_off
            srcb[b][pl.ds(j * 16, 16)] = v
            return carry

        lax.fori_loop(0, EB // 16, adj, 0)

    def issue_gather(b):
        pltpu.async_copy(tab_hbm.at[srcb[b]], rowsb[b], gsems[b])

    def wait_gather(b):
        pltpu.make_async_copy(tab_hbm.at[srcb[b]], rowsb[b], gsems[b]).wait()

    def issue_idx_dyn(i2, b):
        pltpu.async_copy(edge_hbm.at[0, pl.ds(base0 + i2 * EB, EB)], srcb[b],
                         isems[b])
        pltpu.async_copy(edge_hbm.at[1, pl.ds(base0 + i2 * EB, EB)], dstb[b],
                         isems[b])

    if not db:
        def sblk(i, carry):
            issue_idx_dyn(i, 0)
            wait_idx(0)
            adjust(0)
            issue_gather(0)
            wait_gather(0)
            pltpu.sync_copy(rowsb[0], acc_sh.at[dstb[0]], add=True)
            return carry

        lax.fori_loop(0, nblk, sblk, 0)
        return

    issue_idx(0, 0)
    wait_idx(0)
    adjust(0)
    issue_gather(0)
    if nblk > 1:
        issue_idx(1, 1)

    def blk(i, carry):
        def one(b):
            nb = 1 - b

            @pl.when(i + 1 < nblk)
            def _():
                wait_idx(nb)
                adjust(nb)

            wait_gather(b)

            @pl.when(i + 1 < nblk)
            def _():
                issue_gather(nb)

            pltpu.sync_copy(rowsb[b], acc_sh.at[dstb[b]], add=True)

            @pl.when(i + 2 < nblk)
            def _():
                issue_idx_dyn(i + 2, b)

        lax.cond(i % 2 == 0, lambda: one(0), lambda: one(1))
        return carry

    lax.fori_loop(0, nblk, blk, 0)


def _agg1_body(edge_hbm, xd_hbm, zeros_hbm, out_hbm,
               src0, src1, dst0, dst1, rows0, rows1, acc_sh,
               i0, i1, g0, g1):
    c = lax.axis_index("c")
    s = lax.axis_index("s")
    pltpu.sync_copy(zeros_hbm.at[pl.ds(s * RPT, RPT)],
                    acc_sh.at[pl.ds(s * RPT, RPT)])
    plsc.subcore_barrier()
    wid = c * 16 + s
    ept = E // 32
    _edge_loop(EBB, ept // EBB, wid * ept, 1, None, edge_hbm, xd_hbm,
               (src0, src1), (dst0, dst1), (rows0, rows1),
               (i0, i1), (g0, g1), acc_sh, db=True)
    plsc.subcore_barrier()
    pltpu.sync_copy(acc_sh.at[pl.ds(s * RPT, RPT)],
                    out_hbm.at[c, pl.ds(s * RPT, RPT)])


def _sc_agg1(edge_index, xd, zeros16):
    fn = pl.kernel(
        _agg1_body,
        out_type=jax.ShapeDtypeStruct((2, NP, 16), jnp.float32),
        mesh=plsc.VectorSubcoreMesh(**_MESH),
        scratch_types=[
            pltpu.VMEM((EBB,), jnp.int32),
            pltpu.VMEM((EBB,), jnp.int32),
            pltpu.VMEM((EBB,), jnp.int32),
            pltpu.VMEM((EBB,), jnp.int32),
            pltpu.VMEM((EBB, 16), jnp.float32),
            pltpu.VMEM((EBB, 16), jnp.float32),
            pltpu.VMEM_SHARED((NP, 16), jnp.float32),
            pltpu.SemaphoreType.DMA,
            pltpu.SemaphoreType.DMA,
            pltpu.SemaphoreType.DMA,
            pltpu.SemaphoreType.DMA,
        ],
        **_SC_PARAMS,
    )
    return fn(edge_index, xd, zeros16)


def _agg2_body(edge_hbm, u_hbm, zeros_hbm, out_hbm,
               src0, src1, dst0, dst1, rows0, rows1, acc_sh,
               i0, i1, g0, g1):
    c = lax.axis_index("c")
    s = lax.axis_index("s")
    ept = E // 16

    for p in range(2):
        chunk = c * 2 + p
        pltpu.sync_copy(zeros_hbm.at[pl.ds(s * RPT, RPT)],
                        acc_sh.at[pl.ds(s * RPT, RPT)])
        plsc.subcore_barrier()
        _edge_loop(EBC, ept // EBC, s * ept, 4, chunk, edge_hbm,
                   u_hbm, (src0, src1), (dst0, dst1), (rows0, rows1),
                   (i0, i1), (g0, g1), acc_sh)
        plsc.subcore_barrier()
        pltpu.sync_copy(acc_sh.at[pl.ds(s * RPT, RPT)],
                        out_hbm.at[chunk, pl.ds(s * RPT, RPT)])
        plsc.subcore_barrier()


def _sc_agg2(edge_index, u_flat, zeros16):
    fn = pl.kernel(
        _agg2_body,
        out_type=jax.ShapeDtypeStruct((4, NP, 16), jnp.float32),
        mesh=plsc.VectorSubcoreMesh(**_MESH),
        scratch_types=[
            pltpu.VMEM((EBC,), jnp.int32),
            pltpu.VMEM((EBC,), jnp.int32),
            pltpu.VMEM((EBC,), jnp.int32),
            pltpu.VMEM((EBC,), jnp.int32),
            pltpu.VMEM((EBC, 16), jnp.float32),
            pltpu.VMEM((EBC, 16), jnp.float32),
            pltpu.VMEM_SHARED((NP, 16), jnp.float32),
            pltpu.SemaphoreType.DMA,
            pltpu.SemaphoreType.DMA,
            pltpu.SemaphoreType.DMA,
            pltpu.SemaphoreType.DMA,
        ],
        **_SC_PARAMS,
    )
    return fn(edge_index, u_flat, zeros16)


def _tc1_body(degp_ref, x_ref, out_ref):
    d = degp_ref[0] + degp_ref[1] + 1.0
    dinv = lax.rsqrt(d)
    xb = x_ref[...]
    out_ref[...] = jnp.concatenate(
        [xb * dinv, dinv, jnp.zeros((BN1, 12), jnp.float32)], axis=1)


def _tc1(degp, x_pad):
    return pl.pallas_call(
        _tc1_body,
        grid=(NP // BN1,),
        in_specs=[
            pl.BlockSpec((2, BN1, 1), lambda i: (0, i, 0)),
            pl.BlockSpec((BN1, 3), lambda i: (i, 0)),
        ],
        out_specs=pl.BlockSpec((BN1, 16), lambda i: (i, 0)),
        out_shape=jax.ShapeDtypeStruct((NP, 16), jnp.float32),
    )(degp.reshape(2, NP, 1), x_pad)


def _tc2_body(xd_ref, aggp_ref, w1_ref, b1_ref, out_ref):
    xb = xd_ref[...]
    ag = aggp_ref[0] + aggp_ref[1] + xb
    dinv = xb[:, 3:4]
    z = jnp.dot(ag * dinv, w1_ref[...], preferred_element_type=jnp.float32)
    z = z + b1_ref[...]
    h1 = jnp.where(z > 0, z, jnp.exp(z) - 1.0)
    out_ref[...] = h1 * dinv


def _tc2(xd, agg1p, W1p, b1):
    return pl.pallas_call(
        _tc2_body,
        grid=(NP // BN1,),
        in_specs=[
            pl.BlockSpec((BN1, 16), lambda i: (i, 0)),
            pl.BlockSpec((2, BN1, 16), lambda i: (0, i, 0)),
            pl.BlockSpec((16, HID), lambda i: (0, 0)),
            pl.BlockSpec((1, HID), lambda i: (0, 0)),
        ],
        out_specs=pl.BlockSpec((BN1, HID), lambda i: (i, 0)),
        out_shape=jax.ShapeDtypeStruct((NP, HID), jnp.float32),
    )(xd, agg1p, W1p, b1.reshape(1, HID))


def _tc3_body(agg2_ref, u_ref, xd_ref, w2_ref, b2_ref, wm1_ref, bm1_ref,
              wm2_ref, bm2_ref, out_ref):
    u = u_ref[...]
    dinv = xd_ref[:, 3:4]
    z2 = jnp.dot(u * dinv, w2_ref[...], preferred_element_type=jnp.float32)
    for c0 in range(4):
        z2 = z2 + jnp.dot(agg2_ref[c0] * dinv,
                          w2_ref[16 * c0:16 * (c0 + 1), :],
                          preferred_element_type=jnp.float32)
    z2 = z2 + b2_ref[...]
    h2 = jnp.where(z2 > 0, z2, jnp.exp(z2) - 1.0)
    zm = jnp.dot(h2, wm1_ref[...], preferred_element_type=jnp.float32)
    zm = zm + bm1_ref[...]
    hm = jnp.where(zm > 0, zm, jnp.exp(zm) - 1.0)
    H = jnp.dot(hm, wm2_ref[...], preferred_element_type=jnp.float32)
    H = H + bm2_ref[...]
    m = jnp.max(H, axis=1, keepdims=True)
    e = jnp.exp(H - m)
    S = e / jnp.sum(e, axis=1, keepdims=True)
    out_ref[...] = S[:, :K]


def _tc3(agg2, u, xd, W2, b2, Wm1, bm1, Wm2p, bm2p):
    return pl.pallas_call(
        _tc3_body,
        grid=(N // BN3,),
        in_specs=[
            pl.BlockSpec((4, BN3, 16), lambda i: (0, i, 0)),
            pl.BlockSpec((BN3, HID), lambda i: (i, 0)),
            pl.BlockSpec((BN3, 16), lambda i: (i, 0)),
            pl.BlockSpec((HID, HID), lambda i: (0, 0)),
            pl.BlockSpec((1, HID), lambda i: (0, 0)),
            pl.BlockSpec((HID, MLP_HID), lambda i: (0, 0)),
            pl.BlockSpec((1, MLP_HID), lambda i: (0, 0)),
            pl.BlockSpec((MLP_HID, 16), lambda i: (0, 0)),
            pl.BlockSpec((1, 16), lambda i: (0, 0)),
        ],
        out_specs=pl.BlockSpec((BN3, K), lambda i: (i, 0)),
        out_shape=jax.ShapeDtypeStruct((N, K), jnp.float32),
    )(agg2, u, xd, W2, b2.reshape(1, HID), Wm1, bm1.reshape(1, MLP_HID),
      Wm2p, bm2p.reshape(1, 16))


def kernel(x, edge_index, W1, b1, W2, b2, Wm1, bm1, Wm2, bm2):
    x_pad = jnp.zeros((NP, 3), jnp.float32).at[:N].set(x)
    W1p = jnp.zeros((16, HID), jnp.float32).at[:3].set(W1)
    Wm2p = jnp.zeros((MLP_HID, 16), jnp.float32).at[:, :K].set(Wm2)
    bm2p = jnp.concatenate([bm2, jnp.full((1,), -1e30, jnp.float32)])
    zeros1 = jnp.zeros((NP,), jnp.float32)
    zeros16 = jnp.zeros((NP, 16), jnp.float32)
    degp = _sc_deg(edge_index, zeros1)
    xd = _tc1(degp, x_pad)
    agg1p = _sc_agg1(edge_index, xd, zeros16)
    u = _tc2(xd, agg1p, W1p, b1)
    agg2 = _sc_agg2(edge_index, u.reshape(4 * NP, 16), zeros16)
    return _tc3(agg2, u, xd, W2, b2, Wm1, bm1, Wm2p, bm2p)
